# baseline (speedup 1.0000x reference)
import functools

import jax
import jax.numpy as jnp
from jax.experimental import pallas as pl
from jax.experimental.pallas import tpu as pltpu

HIDDEN_DIM = 768
EXPERT_NUM = 64
TOP_K = 8
N_TOKENS = 32768

TILE = 4096


def _router_kernel(x_ref, wt_ref, b_ref, router_ref, weight_ref, idx_ref,
                   mask_ref):
    r = jnp.dot(x_ref[...], wt_ref[...],
                preferred_element_type=jnp.float32) + b_ref[...]
    router_ref[...] = r

    rt = r.T
    e_iota = jax.lax.broadcasted_iota(jnp.int32, (EXPERT_NUM, TILE),
                                      0).astype(jnp.float32)

    vals = rt
    top_vals = []
    top_idx = []
    for _ in range(TOP_K):
        m = jnp.max(vals, axis=0, keepdims=True)
        is_max = vals == m
        idx = jnp.min(jnp.where(is_max, e_iota, float(EXPERT_NUM)), axis=0,
                      keepdims=True)
        top_vals.append(m)
        top_idx.append(idx)
        vals = jnp.where(e_iota == idx, -jnp.inf, vals)

    vals8t = jnp.concatenate(top_vals, axis=0)
    idx8t = jnp.concatenate(top_idx, axis=0)
    idx_ref[...] = idx8t.astype(jnp.int32).T

    e = jnp.exp(vals8t - vals8t[0:1, :])
    weight_ref[...] = (e / jnp.sum(e, axis=0, keepdims=True)).T

    ek_iota = jax.lax.broadcasted_iota(jnp.int32, (EXPERT_NUM, TOP_K, TILE),
                                       0).astype(jnp.float32)
    mask_ref[...] = (ek_iota == idx8t[None, :, :]).astype(jnp.int32)


@functools.partial(jax.jit, static_argnums=())
def kernel(x, gate_w, gate_b):
    wt = gate_w.T
    b2 = gate_b.reshape(1, EXPERT_NUM)
    grid = (N_TOKENS // TILE,)

    out_shapes = (
        jax.ShapeDtypeStruct((N_TOKENS, EXPERT_NUM), jnp.float32),
        jax.ShapeDtypeStruct((N_TOKENS, TOP_K), jnp.float32),
        jax.ShapeDtypeStruct((N_TOKENS, TOP_K), jnp.int32),
        jax.ShapeDtypeStruct((EXPERT_NUM, TOP_K, N_TOKENS), jnp.int32),
    )
    in_specs = [
        pl.BlockSpec((TILE, HIDDEN_DIM), lambda i: (i, 0)),
        pl.BlockSpec((HIDDEN_DIM, EXPERT_NUM), lambda i: (0, 0)),
        pl.BlockSpec((1, EXPERT_NUM), lambda i: (0, 0)),
    ]
    out_specs = (
        pl.BlockSpec((TILE, EXPERT_NUM), lambda i: (i, 0)),
        pl.BlockSpec((TILE, TOP_K), lambda i: (i, 0)),
        pl.BlockSpec((TILE, TOP_K), lambda i: (i, 0)),
        pl.BlockSpec((EXPERT_NUM, TOP_K, TILE), lambda i: (0, 0, i)),
    )
    router, weight, idx, mask = pl.pallas_call(
        _router_kernel,
        grid=grid,
        in_specs=in_specs,
        out_specs=out_specs,
        out_shape=out_shapes,
        compiler_params=pltpu.CompilerParams(
            dimension_semantics=("parallel",),
        ),
    )(x, wt, b2)
    return (router, weight, idx, mask)


# speedup vs baseline: 1.0188x; 1.0188x over previous
import functools

import jax
import jax.numpy as jnp
from jax.experimental import pallas as pl
from jax.experimental.pallas import tpu as pltpu

HIDDEN_DIM = 768
EXPERT_NUM = 64
TOP_K = 8
N_TOKENS = 32768

TILE = 4096


def _router_kernel(x_ref, w_ref, b_ref, router_ref, weight_ref, idx_ref,
                   mask_ref):
    r = jax.lax.dot_general(
        x_ref[...], w_ref[...],
        dimension_numbers=(((1,), (1,)), ((), ())),
        preferred_element_type=jnp.float32) + b_ref[...]
    router_ref[...] = r

    rt = r.T
    e_iota = jax.lax.broadcasted_iota(jnp.int32, (EXPERT_NUM, TILE),
                                      0).astype(jnp.float32)

    vals = rt
    top_vals = []
    top_idx = []
    for _ in range(TOP_K):
        m = jnp.max(vals, axis=0, keepdims=True)
        is_max = vals == m
        idx = jnp.min(jnp.where(is_max, e_iota, float(EXPERT_NUM)), axis=0,
                      keepdims=True)
        top_vals.append(m)
        top_idx.append(idx)
        vals = jnp.where(e_iota == idx, -jnp.inf, vals)

    vals8t = jnp.concatenate(top_vals, axis=0)
    idx8t = jnp.concatenate(top_idx, axis=0)
    idx_ref[...] = idx8t.astype(jnp.int32).T

    e = jnp.exp(vals8t - vals8t[0:1, :])
    weight_ref[...] = (e / jnp.sum(e, axis=0, keepdims=True)).T

    ek_iota = jax.lax.broadcasted_iota(jnp.int32, (EXPERT_NUM, TOP_K, TILE),
                                       0).astype(jnp.float32)
    mask_ref[...] = (ek_iota == idx8t[None, :, :]).astype(jnp.int32)


@functools.partial(jax.jit, static_argnums=())
def kernel(x, gate_w, gate_b):
    b2 = gate_b.reshape(1, EXPERT_NUM)
    grid = (N_TOKENS // TILE,)

    out_shapes = (
        jax.ShapeDtypeStruct((N_TOKENS, EXPERT_NUM), jnp.float32),
        jax.ShapeDtypeStruct((N_TOKENS, TOP_K), jnp.float32),
        jax.ShapeDtypeStruct((N_TOKENS, TOP_K), jnp.int32),
        jax.ShapeDtypeStruct((EXPERT_NUM, TOP_K, N_TOKENS), jnp.int32),
    )
    in_specs = [
        pl.BlockSpec((TILE, HIDDEN_DIM), lambda i: (i, 0)),
        pl.BlockSpec((EXPERT_NUM, HIDDEN_DIM), lambda i: (0, 0)),
        pl.BlockSpec((1, EXPERT_NUM), lambda i: (0, 0)),
    ]
    out_specs = (
        pl.BlockSpec((TILE, EXPERT_NUM), lambda i: (i, 0)),
        pl.BlockSpec((TILE, TOP_K), lambda i: (i, 0)),
        pl.BlockSpec((TILE, TOP_K), lambda i: (i, 0)),
        pl.BlockSpec((EXPERT_NUM, TOP_K, TILE), lambda i: (0, 0, i)),
    )
    router, weight, idx, mask = pl.pallas_call(
        _router_kernel,
        grid=grid,
        in_specs=in_specs,
        out_specs=out_specs,
        out_shape=out_shapes,
        compiler_params=pltpu.CompilerParams(
            dimension_semantics=("parallel",),
        ),
    )(x, gate_w, b2)
    return (router, weight, idx, mask)
